# Initial kernel scaffold; baseline (speedup 1.0000x reference)
import jax
import jax.numpy as jnp
from jax.experimental import pallas as pl


def kernel(x, adj, W, b):
    raise NotImplementedError("write your pallas kernel here")


# speedup vs baseline: 1.3743x; 1.3743x over previous
import functools

import jax
import jax.numpy as jnp
from jax.experimental import pallas as pl
from jax.experimental.pallas import tpu as pltpu

B, H, N, F_IN, F_OUT = 8, 12, 1024, 192, 192
DH = F_OUT // H
BN = 128


def _gcn_body(x_ref, adj_ref, w_ref, b_ref, h_ref, adjc_ref, support_ref):
    i = pl.program_id(1)

    @pl.when(i == 0)
    def _():
        support_ref[...] = jnp.dot(
            x_ref[0].astype(jnp.bfloat16),
            w_ref[...].astype(jnp.bfloat16),
            preferred_element_type=jnp.float32,
        )

    adjc_ref[...] = adj_ref[...]

    parts = []
    for h in range(H):
        a = adj_ref[0, h].astype(jnp.bfloat16)
        s = support_ref[:, h * DH:(h + 1) * DH]
        parts.append(
            jnp.dot(a, s.astype(jnp.bfloat16),
                    preferred_element_type=jnp.float32)
        )
    acc = jnp.concatenate(parts, axis=-1)
    h_ref[0] = jax.nn.gelu(acc + b_ref[...])


@functools.partial(jax.jit, static_argnames=())
def kernel(x, adj, W, b):
    b2 = b.reshape(1, F_OUT)
    grid = (B, N // BN)
    h_out, adjc = pl.pallas_call(
        _gcn_body,
        grid=grid,
        in_specs=[
            pl.BlockSpec((1, N, F_IN), lambda bi, i: (bi, 0, 0)),
            pl.BlockSpec((1, H, BN, N), lambda bi, i: (bi, 0, i, 0)),
            pl.BlockSpec((F_IN, F_OUT), lambda bi, i: (0, 0)),
            pl.BlockSpec((1, F_OUT), lambda bi, i: (0, 0)),
        ],
        out_specs=[
            pl.BlockSpec((1, BN, F_OUT), lambda bi, i: (bi, i, 0)),
            pl.BlockSpec((1, H, BN, N), lambda bi, i: (bi, 0, i, 0)),
        ],
        out_shape=[
            jax.ShapeDtypeStruct((B, N, F_OUT), jnp.float32),
            jax.ShapeDtypeStruct((B, H, N, N), jnp.float32),
        ],
        scratch_shapes=[pltpu.VMEM((N, F_OUT), jnp.float32)],
    )(x, adj, W, b2)
    return h_out, adjc.reshape(B * H, N, N)


# speedup vs baseline: 1.3876x; 1.0097x over previous
import functools

import jax
import jax.numpy as jnp
from jax.experimental import pallas as pl
from jax.experimental.pallas import tpu as pltpu

B, H, N, F_IN, F_OUT = 8, 12, 1024, 192, 192
DH = F_OUT // H
BN = 256


def _gcn_body(x_ref, adj_ref, w_ref, b_ref, h_ref, adjc_ref, support_ref):
    i = pl.program_id(1)

    @pl.when(i == 0)
    def _():
        support_ref[...] = jnp.dot(
            x_ref[0].astype(jnp.bfloat16),
            w_ref[...].astype(jnp.bfloat16),
            preferred_element_type=jnp.float32,
        )

    adjc_ref[...] = adj_ref[...]

    parts = []
    for h in range(H):
        a = adj_ref[0, h].astype(jnp.bfloat16)
        s = support_ref[:, h * DH:(h + 1) * DH]
        parts.append(
            jnp.dot(a, s.astype(jnp.bfloat16),
                    preferred_element_type=jnp.float32)
        )
    acc = jnp.concatenate(parts, axis=-1)
    h_ref[0] = jax.nn.gelu(acc + b_ref[...])


@functools.partial(jax.jit, static_argnames=())
def kernel(x, adj, W, b):
    b2 = b.reshape(1, F_OUT)
    grid = (B, N // BN)
    h_out, adjc = pl.pallas_call(
        _gcn_body,
        grid=grid,
        in_specs=[
            pl.BlockSpec((1, N, F_IN), lambda bi, i: (bi, 0, 0)),
            pl.BlockSpec((1, H, BN, N), lambda bi, i: (bi, 0, i, 0)),
            pl.BlockSpec((F_IN, F_OUT), lambda bi, i: (0, 0)),
            pl.BlockSpec((1, F_OUT), lambda bi, i: (0, 0)),
        ],
        out_specs=[
            pl.BlockSpec((1, BN, F_OUT), lambda bi, i: (bi, i, 0)),
            pl.BlockSpec((1, H, BN, N), lambda bi, i: (bi, 0, i, 0)),
        ],
        out_shape=[
            jax.ShapeDtypeStruct((B, N, F_OUT), jnp.float32),
            jax.ShapeDtypeStruct((B, H, N, N), jnp.float32),
        ],
        scratch_shapes=[pltpu.VMEM((N, F_OUT), jnp.float32)],
    )(x, adj, W, b2)
    return h_out, adjc.reshape(B * H, N, N)
